# Initial kernel scaffold; baseline (speedup 1.0000x reference)
import jax
import jax.numpy as jnp
from jax.experimental import pallas as pl


def kernel(positives, negatives, r_head_base_points, r_head_widths, r_head_size_scales, r_tail_base_points, r_tail_widths, r_tail_size_scales, entity_bases, entity_bumps):
    raise NotImplementedError("write your pallas kernel here")


# speedup vs baseline: 1.8420x; 1.8420x over previous
import functools

import jax
import jax.numpy as jnp
from jax import lax
from jax.experimental import pallas as pl
from jax.experimental.pallas import tpu as pltpu
from jax.experimental.pallas import tpu_sc as plsc

E_DIM = 512
N_REL = 600
NC = 2
NS = 16
NW = NC * NS
LANES = 16


def _box_body(rhb, rhw, rhs, rtb, rtw, rts, out):
    def half(base_ref, width_ref, scale_ref):
        w = width_ref[...]
        step2 = jnp.abs(w) + 1e-8
        norm_volume = jnp.exp(jnp.mean(jnp.log(step2), axis=1, keepdims=True))
        wn = w / norm_volume
        sc = scale_ref[...]
        s = jnp.where(sc > 0, sc, jnp.exp(sc) - 1.0) + 1.0
        d = wn * s
        b = base_ref[...]
        c1 = b + d
        c2 = b - d
        return jnp.maximum(c1, c2), jnp.minimum(c1, c2)

    hu, hl = half(rhb, rhw, rhs)
    tu, tl = half(rtb, rtw, rts)
    out[:, 0 * E_DIM:1 * E_DIM] = hu
    out[:, 1 * E_DIM:2 * E_DIM] = hl
    out[:, 2 * E_DIM:3 * E_DIM] = tu
    out[:, 3 * E_DIM:4 * E_DIM] = tl


def _box_tables(rhb, rhw, rhs, rtb, rtw, rts):
    rows = 120
    grid = N_REL // rows
    full = lambda i: (i, 0)
    return pl.pallas_call(
        _box_body,
        grid=(grid,),
        in_specs=[
            pl.BlockSpec((rows, E_DIM), full),
            pl.BlockSpec((rows, E_DIM), full),
            pl.BlockSpec((rows, 1), full),
            pl.BlockSpec((rows, E_DIM), full),
            pl.BlockSpec((rows, E_DIM), full),
            pl.BlockSpec((rows, 1), full),
        ],
        out_specs=pl.BlockSpec((rows, 4 * E_DIM), full),
        out_shape=jax.ShapeDtypeStruct((N_REL, 4 * E_DIM), jnp.float32),
    )(rhb, rhw, rhs, rtb, rtw, rts)


def _mesh():
    return plsc.VectorSubcoreMesh(
        core_axis_name="c", subcore_axis_name="s", num_cores=NC, num_subcores=NS
    )


@functools.lru_cache(maxsize=None)
def _rel_gather(T, CH):
    c = T // NW
    nch = c // CH

    @functools.partial(
        pl.kernel,
        mesh=_mesh(),
        out_type=jax.ShapeDtypeStruct((T, 4 * E_DIM), jnp.float32),
        scratch_types=[
            pltpu.VMEM((c,), jnp.int32),
            pltpu.VMEM((CH, 4 * E_DIM), jnp.float32),
            pltpu.SemaphoreType.DMA,
        ],
    )
    def k(rid_hbm, boxes_hbm, out_hbm, ridx, rows_v, sem):
        wid = lax.axis_index("s") * NC + lax.axis_index("c")
        base = wid * c
        pltpu.sync_copy(rid_hbm.at[pl.ds(base, c)], ridx)

        def chunk(i, _):
            off = i * CH
            pltpu.async_copy(
                boxes_hbm.at[ridx.at[pl.ds(off, CH)]], rows_v, sem
            ).wait()
            pltpu.sync_copy(rows_v, out_hbm.at[pl.ds(base + off, CH)])
            return 0

        lax.fori_loop(0, nch, chunk, 0)

    return k


@functools.lru_cache(maxsize=None)
def _ent_gather(T, CH):
    c = T // NW
    nch = c // CH
    W2 = 2 * E_DIM
    KSTEPS = E_DIM // LANES

    @functools.partial(
        pl.kernel,
        mesh=_mesh(),
        out_type=jax.ShapeDtypeStruct((T, W2), jnp.float32),
        scratch_types=[
            pltpu.VMEM((c,), jnp.int32),
            pltpu.VMEM((c,), jnp.int32),
            pltpu.VMEM((CH, W2), jnp.float32),
            pltpu.VMEM((CH, W2), jnp.float32),
            pltpu.VMEM((CH, W2), jnp.float32),
            pltpu.SemaphoreType.DMA,
        ],
    )
    def k(hid_hbm, tid_hbm, ent2_hbm, out_hbm, hidx, tidx, hrows, trows, erows, sem):
        wid = lax.axis_index("s") * NC + lax.axis_index("c")
        base = wid * c
        pltpu.sync_copy(hid_hbm.at[pl.ds(base, c)], hidx)
        pltpu.sync_copy(tid_hbm.at[pl.ds(base, c)], tidx)

        def chunk(i, _):
            off = i * CH
            cph = pltpu.async_copy(ent2_hbm.at[hidx.at[pl.ds(off, CH)]], hrows, sem)
            cpt = pltpu.async_copy(ent2_hbm.at[tidx.at[pl.ds(off, CH)]], trows, sem)
            cph.wait()
            cpt.wait()

            def row(j, _):
                def vec(kk, _):
                    lo = kk * LANES
                    hi = E_DIM + lo
                    erows[j, pl.ds(lo, LANES)] = (
                        hrows[j, pl.ds(lo, LANES)] + trows[j, pl.ds(hi, LANES)]
                    )
                    erows[j, pl.ds(hi, LANES)] = (
                        trows[j, pl.ds(lo, LANES)] + hrows[j, pl.ds(hi, LANES)]
                    )
                    return 0

                lax.fori_loop(0, KSTEPS, vec, 0)
                return 0

            lax.fori_loop(0, CH, row, 0)
            pltpu.sync_copy(erows, out_hbm.at[pl.ds(base + off, CH)])
            return 0

        lax.fori_loop(0, nch, chunk, 0)

    return k


def kernel(positives, negatives, r_head_base_points, r_head_widths,
           r_head_size_scales, r_tail_base_points, r_tail_widths,
           r_tail_size_scales, entity_bases, entity_bumps):
    boxes = _box_tables(r_head_base_points, r_head_widths, r_head_size_scales,
                        r_tail_base_points, r_tail_widths, r_tail_size_scales)
    ent2 = jnp.concatenate([entity_bases, entity_bumps], axis=1)

    def ids(tuples, col):
        return tuples[:, col, :].reshape(-1).astype(jnp.int32)

    outs = []
    for tuples in (positives, negatives):
        n, _, b = tuples.shape
        T = n * b
        h = ids(tuples, 0)
        r = ids(tuples, 1)
        t = ids(tuples, 2)
        ent = _ent_gather(T, 8)(h, t, ent2)
        rel = _rel_gather(T, 8)(r, boxes)
        outs.append(ent.reshape(n, b, 2, E_DIM))
        outs.append(rel.reshape(n, b, 2, 2, E_DIM))

    p_ent, p_rel, n_ent, n_rel = outs
    return (p_ent, p_rel, n_ent, n_rel)


# speedup vs baseline: 2.7145x; 1.4737x over previous
import functools

import jax
import jax.numpy as jnp
from jax import lax
from jax.experimental import pallas as pl
from jax.experimental.pallas import tpu as pltpu
from jax.experimental.pallas import tpu_sc as plsc

E_DIM = 512
N_REL = 600
NC = 2
NS = 16
NW = NC * NS
LANES = 16
P_T = 512
N_T = 16384
CH = 8
S = 4


def _box_body(rhb, rhw, rhs, rtb, rtw, rts, out):
    def half(base_ref, width_ref, scale_ref):
        w = width_ref[...]
        step2 = jnp.abs(w) + 1e-8
        norm_volume = jnp.exp(jnp.mean(jnp.log(step2), axis=1, keepdims=True))
        wn = w / norm_volume
        sc = scale_ref[...]
        s = jnp.where(sc > 0, sc, jnp.exp(sc) - 1.0) + 1.0
        d = wn * s
        b = base_ref[...]
        c1 = b + d
        c2 = b - d
        return jnp.maximum(c1, c2), jnp.minimum(c1, c2)

    hu, hl = half(rhb, rhw, rhs)
    tu, tl = half(rtb, rtw, rts)
    out[:, 0 * E_DIM:1 * E_DIM] = hu
    out[:, 1 * E_DIM:2 * E_DIM] = hl
    out[:, 2 * E_DIM:3 * E_DIM] = tu
    out[:, 3 * E_DIM:4 * E_DIM] = tl


def _box_tables(rhb, rhw, rhs, rtb, rtw, rts):
    rows = 120
    grid = N_REL // rows
    full = lambda i: (i, 0)
    return pl.pallas_call(
        _box_body,
        grid=(grid,),
        in_specs=[
            pl.BlockSpec((rows, E_DIM), full),
            pl.BlockSpec((rows, E_DIM), full),
            pl.BlockSpec((rows, 1), full),
            pl.BlockSpec((rows, E_DIM), full),
            pl.BlockSpec((rows, E_DIM), full),
            pl.BlockSpec((rows, 1), full),
        ],
        out_specs=pl.BlockSpec((rows, 4 * E_DIM), full),
        out_shape=jax.ShapeDtypeStruct((N_REL, 4 * E_DIM), jnp.float32),
    )(rhb, rhw, rhs, rtb, rtw, rts)


_CP = P_T // NW
_CN = N_T // NW
_NPC = _CP // CH
_NNC = _CN // CH
_TOTAL = _NPC + _NNC


def _mesh():
    return plsc.VectorSubcoreMesh(
        core_axis_name="c", subcore_axis_name="s", num_cores=NC, num_subcores=NS
    )


def _worker_id():
    return lax.axis_index("s") * NC + lax.axis_index("c")


@functools.lru_cache(maxsize=None)
def _ent_kernel():
    W2 = 2 * E_DIM

    scratch = (
        [pltpu.VMEM((_CP,), jnp.int32)] * 2
        + [pltpu.VMEM((_CN,), jnp.int32)] * 2
        + [pltpu.VMEM((CH, W2), jnp.float32)] * (2 * S)
        + [pltpu.SemaphoreType.DMA] * (2 * S)
    )

    @functools.partial(
        pl.kernel,
        mesh=_mesh(),
        out_type=(
            jax.ShapeDtypeStruct((P_T, W2), jnp.float32),
            jax.ShapeDtypeStruct((N_T, W2), jnp.float32),
        ),
        scratch_types=scratch,
    )
    def k(hp_hbm, tp_hbm, hn_hbm, tn_hbm, ent2_hbm, outp_hbm, outn_hbm, *sc):
        hidx_p, tidx_p, hidx_n, tidx_n = sc[0:4]
        hb = sc[4:4 + S]
        tb = sc[4 + S:4 + 2 * S]
        gsem = sc[4 + 2 * S:4 + 3 * S]
        wsem = sc[4 + 3 * S:4 + 4 * S]

        wid = _worker_id()
        pltpu.sync_copy(hp_hbm.at[pl.ds(wid * _CP, _CP)], hidx_p)
        pltpu.sync_copy(tp_hbm.at[pl.ds(wid * _CP, _CP)], tidx_p)
        pltpu.sync_copy(hn_hbm.at[pl.ds(wid * _CN, _CN)], hidx_n)
        pltpu.sync_copy(tn_hbm.at[pl.ds(wid * _CN, _CN)], tidx_n)

        def fire_pos(j, s):
            off = j * CH
            pltpu.async_copy(ent2_hbm.at[hidx_p.at[pl.ds(off, CH)]], hb[s], gsem[s])
            pltpu.async_copy(ent2_hbm.at[tidx_p.at[pl.ds(off, CH)]], tb[s], gsem[s])

        def fire_neg(jj, s):
            off = jj * CH
            pltpu.async_copy(ent2_hbm.at[hidx_n.at[pl.ds(off, CH)]], hb[s], gsem[s])
            pltpu.async_copy(ent2_hbm.at[tidx_n.at[pl.ds(off, CH)]], tb[s], gsem[s])

        def wait_g(s):
            pltpu.make_async_copy(outn_hbm.at[pl.ds(0, CH)], hb[s], gsem[s]).wait()
            pltpu.make_async_copy(outn_hbm.at[pl.ds(0, CH)], tb[s], gsem[s]).wait()

        def compute(s):
            def row(i, _):
                def vec(kk, _):
                    lo = kk * LANES
                    hi = E_DIM + lo
                    a = hb[s][i, pl.ds(lo, LANES)]
                    bv = tb[s][i, pl.ds(hi, LANES)]
                    cv = tb[s][i, pl.ds(lo, LANES)]
                    dv = hb[s][i, pl.ds(hi, LANES)]
                    hb[s][i, pl.ds(lo, LANES)] = a + bv
                    hb[s][i, pl.ds(hi, LANES)] = cv + dv
                    return 0

                lax.fori_loop(0, E_DIM // LANES, vec, 0, unroll=4)
                return 0

            lax.fori_loop(0, CH, row, 0)

        def wr_pos(j, s):
            pltpu.async_copy(hb[s], outp_hbm.at[pl.ds(wid * _CP + j * CH, CH)], wsem[s])

        def wr_neg(jj, s):
            pltpu.async_copy(hb[s], outn_hbm.at[pl.ds(wid * _CN + jj * CH, CH)], wsem[s])

        def drain_w(s):
            pltpu.make_async_copy(hb[s], outn_hbm.at[pl.ds(0, CH)], wsem[s]).wait()

        fire_pos(0, 0)
        fire_pos(1, 1)
        for j in (0, 1):
            wait_g(j)
            compute(j)
            wr_pos(j, j)
            fire_neg(j, (j + 2) % S)
        for j in (2, 3):
            wait_g(j)
            compute(j)
            wr_neg(j - 2, j)
            drain_w((j + 2) % S)
            fire_neg(j, (j + 2) % S)
        def group(g, _):
            for b in range(S):
                jj = 4 * g + b - 2
                wait_g(b)
                compute(b)
                wr_neg(jj, b)
                nxt = (b + 2) % S

                @pl.when(jj + 2 < _NNC)
                def _():
                    drain_w(nxt)
                    fire_neg(jj + 2, nxt)

                _ = _
            return 0

        lax.fori_loop(1, _TOTAL // S, group, 0)
        for j, s in ((64, 0), (65, 1)):
            wait_g(s)
            compute(s)
            wr_neg(j - 2, s)
        for s in (2, 3, 0, 1):
            drain_w(s)

    return k


@functools.lru_cache(maxsize=None)
def _rel_kernel():
    W4 = 4 * E_DIM

    scratch = (
        [pltpu.VMEM((_CP,), jnp.int32)]
        + [pltpu.VMEM((_CN,), jnp.int32)]
        + [pltpu.VMEM((CH, W4), jnp.float32)] * S
        + [pltpu.SemaphoreType.DMA] * (2 * S)
    )

    @functools.partial(
        pl.kernel,
        mesh=_mesh(),
        out_type=(
            jax.ShapeDtypeStruct((P_T, W4), jnp.float32),
            jax.ShapeDtypeStruct((N_T, W4), jnp.float32),
        ),
        scratch_types=scratch,
    )
    def k(rp_hbm, rn_hbm, boxes_hbm, outp_hbm, outn_hbm, *sc):
        ridx_p, ridx_n = sc[0:2]
        rb = sc[2:2 + S]
        gsem = sc[2 + S:2 + 2 * S]
        wsem = sc[2 + 2 * S:2 + 3 * S]

        wid = _worker_id()
        pltpu.sync_copy(rp_hbm.at[pl.ds(wid * _CP, _CP)], ridx_p)
        pltpu.sync_copy(rn_hbm.at[pl.ds(wid * _CN, _CN)], ridx_n)

        def fire_pos(j, s):
            pltpu.async_copy(boxes_hbm.at[ridx_p.at[pl.ds(j * CH, CH)]], rb[s], gsem[s])

        def fire_neg(jj, s):
            pltpu.async_copy(boxes_hbm.at[ridx_n.at[pl.ds(jj * CH, CH)]], rb[s], gsem[s])

        def wait_g(s):
            pltpu.make_async_copy(outn_hbm.at[pl.ds(0, CH)], rb[s], gsem[s]).wait()

        def wr_pos(j, s):
            pltpu.async_copy(rb[s], outp_hbm.at[pl.ds(wid * _CP + j * CH, CH)], wsem[s])

        def wr_neg(jj, s):
            pltpu.async_copy(rb[s], outn_hbm.at[pl.ds(wid * _CN + jj * CH, CH)], wsem[s])

        def drain_w(s):
            pltpu.make_async_copy(rb[s], outn_hbm.at[pl.ds(0, CH)], wsem[s]).wait()

        fire_pos(0, 0)
        fire_pos(1, 1)
        for j in (0, 1):
            wait_g(j)
            wr_pos(j, j)
            fire_neg(j, (j + 2) % S)
        for j in (2, 3):
            wait_g(j)
            wr_neg(j - 2, j)
            drain_w((j + 2) % S)
            fire_neg(j, (j + 2) % S)

        def group(g, _):
            for b in range(S):
                jj = 4 * g + b - 2
                wait_g(b)
                wr_neg(jj, b)
                nxt = (b + 2) % S

                @pl.when(jj + 2 < _NNC)
                def _():
                    drain_w(nxt)
                    fire_neg(jj + 2, nxt)

                _ = _
            return 0

        lax.fori_loop(1, _TOTAL // S, group, 0)
        for j, s in ((64, 0), (65, 1)):
            wait_g(s)
            wr_neg(j - 2, s)
        for s in (2, 3, 0, 1):
            drain_w(s)

    return k


def kernel(positives, negatives, r_head_base_points, r_head_widths,
           r_head_size_scales, r_tail_base_points, r_tail_widths,
           r_tail_size_scales, entity_bases, entity_bumps):
    boxes = _box_tables(r_head_base_points, r_head_widths, r_head_size_scales,
                        r_tail_base_points, r_tail_widths, r_tail_size_scales)
    ent2 = jnp.concatenate([entity_bases, entity_bumps], axis=1)

    def ids(tuples, col):
        return tuples[:, col, :].reshape(-1).astype(jnp.int32)

    hp, rp, tp = ids(positives, 0), ids(positives, 1), ids(positives, 2)
    hn, rn, tn = ids(negatives, 0), ids(negatives, 1), ids(negatives, 2)

    pe, ne = _ent_kernel()(hp, tp, hn, tn, ent2)
    pr, nr = _rel_kernel()(rp, rn, boxes)

    p_ent = pe.reshape(1, P_T, 2, E_DIM)
    n_ent = ne.reshape(32, P_T, 2, E_DIM)
    p_rel = pr.reshape(1, P_T, 2, 2, E_DIM)
    n_rel = nr.reshape(32, P_T, 2, 2, E_DIM)
    return (p_ent, p_rel, n_ent, n_rel)
